# Initial kernel scaffold; baseline (speedup 1.0000x reference)
import jax
import jax.numpy as jnp
from jax.experimental import pallas as pl


def kernel(pred_scores, pred_boxes):
    raise NotImplementedError("write your pallas kernel here")


# speedup vs baseline: 9.9431x; 9.9431x over previous
import jax
import jax.numpy as jnp
from jax.experimental import pallas as pl
from jax.experimental.pallas import tpu as pltpu

_BIG = 2**30
_K = 100


def _pick_chunk(n):
    for r in (160, 80, 40, 8, 1):
        if n % r == 0:
            return r
    return 1


def _make_body(N, C, R, T, K):
    neg_inf = float("-inf")

    def body(x_ref, bx_ref, os_ref, ol_ref, ob_ref, cmv_ref, cmi_ref):
        def chunk_stats(t):
            blk = x_ref[0, pl.ds(t * R, R), :]
            bm = jnp.max(blk, axis=0, keepdims=True)
            rowi = jax.lax.broadcasted_iota(jnp.int32, (R, C), 0)
            coli = jax.lax.broadcasted_iota(jnp.int32, (R, C), 1)
            flat = (t * R + rowi) * C + coli
            cand = jnp.where(blk == bm, flat, _BIG)
            bi = jnp.min(cand, axis=0, keepdims=True)
            return bm, bi

        def init_t(t, _):
            bm, bi = chunk_stats(t)
            cmv_ref[pl.ds(t, 1), :] = bm
            cmi_ref[pl.ds(t, 1), :] = bi
            return 0

        jax.lax.fori_loop(0, T, init_t, 0)

        def extract(k, _):
            cv = cmv_ref[...]
            m = jnp.max(cv)
            fi = jnp.min(jnp.where(cv == m, cmi_ref[...], _BIG))
            row = fi // C
            col = fi - row * C
            t = row // R

            os_ref[0, pl.ds(k, 1), :] = m[None, None]
            ol_ref[0, pl.ds(k, 1), :] = col[None, None]
            ob_ref[0, pl.ds(k, 1), :] = bx_ref[0, pl.ds(row, 1), :]

            rowv = x_ref[0, pl.ds(row, 1), :]
            li = jax.lax.broadcasted_iota(jnp.int32, (1, C), 1)
            x_ref[0, pl.ds(row, 1), :] = jnp.where(li == col, neg_inf, rowv)

            bm, bi = chunk_stats(t)
            cmv_ref[pl.ds(t, 1), :] = bm
            cmi_ref[pl.ds(t, 1), :] = bi
            return 0

        jax.lax.fori_loop(0, K, extract, 0)

    return body


def kernel(pred_scores, pred_boxes):
    B, N, C = pred_scores.shape
    R = _pick_chunk(N)
    T = N // R
    K = _K

    grid = (B,)
    s3, l3, b3 = pl.pallas_call(
        _make_body(N, C, R, T, K),
        grid=grid,
        in_specs=[
            pl.BlockSpec((1, N, C), lambda b: (b, 0, 0)),
            pl.BlockSpec((1, N, 4), lambda b: (b, 0, 0)),
        ],
        out_specs=[
            pl.BlockSpec((1, K, 1), lambda b: (b, 0, 0)),
            pl.BlockSpec((1, K, 1), lambda b: (b, 0, 0)),
            pl.BlockSpec((1, K, 4), lambda b: (b, 0, 0)),
        ],
        out_shape=[
            jax.ShapeDtypeStruct((B, K, 1), jnp.float32),
            jax.ShapeDtypeStruct((B, K, 1), jnp.int32),
            jax.ShapeDtypeStruct((B, K, 4), jnp.float32),
        ],
        scratch_shapes=[
            pltpu.VMEM((T, C), jnp.float32),
            pltpu.VMEM((T, C), jnp.int32),
        ],
    )(pred_scores, pred_boxes)

    return s3[:, :, 0], l3[:, :, 0], b3
